# Initial kernel scaffold; baseline (speedup 1.0000x reference)
import jax
import jax.numpy as jnp
from jax.experimental import pallas as pl


def kernel(edge_index, values, embs):
    raise NotImplementedError("write your pallas kernel here")


# speedup vs baseline: 1.9155x; 1.9155x over previous
import functools

import jax
import jax.numpy as jnp
from jax import lax
from jax.experimental import pallas as pl
from jax.experimental.pallas import tpu as pltpu
from jax.experimental.pallas import tpu_sc as plsc

N = 10000
E = 320000
D = 128
DH = D // 2
LEAKY = 0.2

NS = 16
CH = 80
EPT = E // NS
NCHUNK = EPT // CH
SB = 624
REM = N - NS * SB


def _scale_rows(buf, vals, nrows):
    def body(t, _):
        vvec = vals[pl.ds(t * 16, 16)]
        base = t * 16
        for lane in range(16):
            v = vvec[lane]
            for j in range(DH // 16):
                sl = pl.ds(j * 16, 16)
                buf[base + lane, sl] = buf[base + lane, sl] * v
        return 0
    lax.fori_loop(0, nrows // 16, body, 0)


def _hgcn_body(rows_hbm, cols_hbm, vals_hbm, embs2_hbm, out2_hbm,
               h_sp, o_sp, idx_v, rows_v, cols_v, vals_v, gbuf, obuf, sem):
    c = lax.axis_index("c")
    s = lax.axis_index("s")
    ebase = s * EPT
    rbase = s * SB

    def zbody(i, _):
        zero = jnp.zeros((16,), jnp.float32)
        for j in range(DH // 16):
            obuf[i, pl.ds(j * 16, 16)] = zero
        return 0
    lax.fori_loop(0, SB, zbody, 0)
    pltpu.sync_copy(obuf.at[pl.ds(0, SB)], h_sp.at[pl.ds(rbase, SB)])
    pltpu.sync_copy(obuf.at[pl.ds(0, SB)], o_sp.at[pl.ds(rbase, SB)])
    @pl.when(s == NS - 1)
    def _():
        pltpu.sync_copy(obuf.at[pl.ds(0, REM)], h_sp.at[pl.ds(NS * SB, REM)])
        pltpu.sync_copy(obuf.at[pl.ds(0, REM)], o_sp.at[pl.ds(NS * SB, REM)])
    plsc.subcore_barrier()

    cN = c * N

    def hop1(g, _):
        eo = pl.ds(ebase + g * CH, CH)
        pltpu.sync_copy(rows_hbm.at[eo], rows_v)
        pltpu.sync_copy(cols_hbm.at[eo], cols_v)
        pltpu.sync_copy(vals_hbm.at[eo], vals_v)
        for j in range(CH // 16):
            sl = pl.ds(j * 16, 16)
            idx_v[sl] = rows_v[sl] + cN
        pltpu.async_copy(embs2_hbm.at[idx_v], gbuf, sem).wait()
        _scale_rows(gbuf, vals_v, CH)
        pltpu.sync_copy(gbuf, h_sp.at[cols_v], add=True)
        return 0
    lax.fori_loop(0, NCHUNK, hop1, 0)
    plsc.subcore_barrier()

    def hop2(g, _):
        eo = pl.ds(ebase + g * CH, CH)
        pltpu.sync_copy(rows_hbm.at[eo], rows_v)
        pltpu.sync_copy(cols_hbm.at[eo], cols_v)
        pltpu.sync_copy(vals_hbm.at[eo], vals_v)
        pltpu.async_copy(h_sp.at[cols_v], gbuf, sem).wait()
        _scale_rows(gbuf, vals_v, CH)
        pltpu.sync_copy(gbuf, o_sp.at[rows_v], add=True)
        return 0
    lax.fori_loop(0, NCHUNK, hop2, 0)
    plsc.subcore_barrier()

    def _leaky(nrows):
        def lbody(i, _):
            for j in range(DH // 16):
                sl = pl.ds(j * 16, 16)
                x = obuf[i, sl]
                obuf[i, sl] = jnp.where(x >= 0, x, x * LEAKY)
            return 0
        lax.fori_loop(0, nrows, lbody, 0)

    pltpu.sync_copy(o_sp.at[pl.ds(rbase, SB)], obuf.at[pl.ds(0, SB)])
    _leaky(SB)
    pltpu.sync_copy(obuf.at[pl.ds(0, SB)], out2_hbm.at[pl.ds(cN + rbase, SB)])
    @pl.when(s == NS - 1)
    def _():
        pltpu.sync_copy(o_sp.at[pl.ds(NS * SB, REM)], obuf.at[pl.ds(0, REM)])
        _leaky(REM)
        pltpu.sync_copy(obuf.at[pl.ds(0, REM)],
                        out2_hbm.at[pl.ds(cN + NS * SB, REM)])


@jax.jit
def _hgcn_sc(rows, cols, vals, embs2):
    mesh = plsc.VectorSubcoreMesh(core_axis_name="c", subcore_axis_name="s")
    return pl.kernel(
        _hgcn_body,
        out_type=jax.ShapeDtypeStruct((2 * N, DH), jnp.float32),
        mesh=mesh,
        scratch_types=[
            pltpu.VMEM_SHARED((N, DH), jnp.float32),
            pltpu.VMEM_SHARED((N, DH), jnp.float32),
            pltpu.VMEM((CH,), jnp.int32),
            pltpu.VMEM((CH,), jnp.int32),
            pltpu.VMEM((CH,), jnp.int32),
            pltpu.VMEM((CH,), jnp.float32),
            pltpu.VMEM((CH, DH), jnp.float32),
            pltpu.VMEM((SB, DH), jnp.float32),
            pltpu.SemaphoreType.DMA,
        ],
        compiler_params=pltpu.CompilerParams(use_tc_tiling_on_sc=False),
    )(rows, cols, vals, embs2)


def kernel(edge_index, values, embs):
    rows = edge_index[0].astype(jnp.int32)
    cols = edge_index[1].astype(jnp.int32)
    embs2 = jnp.concatenate([embs[:, :DH], embs[:, DH:]], axis=0)
    out2 = _hgcn_sc(rows, cols, values, embs2)
    return jnp.concatenate([out2[:N], out2[N:]], axis=1)


# speedup vs baseline: 9.0314x; 4.7149x over previous
import functools

import jax
import jax.numpy as jnp
from jax import lax
from jax.experimental import pallas as pl
from jax.experimental.pallas import tpu as pltpu
from jax.experimental.pallas import tpu_sc as plsc

N = 10000
E = 320000
D = 128
DH = D // 2
LEAKY = 0.2

NS = 16
CH = 80
EPT = E // NS
NCHUNK = EPT // CH
CPB = 50
NB = NCHUNK // CPB
SB = 624
REM = N - NS * SB
OB = 48


def _hgcn_body(rows_hbm, cols_hbm, vals_hbm, embs2_hbm, out2_hbm,
               h_sp, o_sp,
               rows_vm, cols_vm, vals_vm,
               ib0, ib1, gb0, gb1, sb0, sb1, obuf,
               gsem0, gsem1, ssem0, ssem1):
    c = lax.axis_index("c")
    s = lax.axis_index("s")
    cN = c * N
    ibuf = (ib0, ib1)
    gbuf = (gb0, gb1)
    sbuf = (sb0, sb1)
    gsem = (gsem0, gsem1)
    ssem = (ssem0, ssem1)

    def zbody(i, _):
        zero = jnp.zeros((16,), jnp.float32)
        for j in range(DH // 16):
            obuf[i, pl.ds(j * 16, 16)] = zero
        return 0
    lax.fori_loop(0, OB, zbody, 0)
    rbase = s * SB
    for k in range(SB // OB):
        pltpu.sync_copy(obuf, h_sp.at[pl.ds(rbase + k * OB, OB)])
        pltpu.sync_copy(obuf, o_sp.at[pl.ds(rbase + k * OB, OB)])
    @pl.when(s == NS - 1)
    def _():
        pltpu.sync_copy(obuf.at[pl.ds(0, REM)], h_sp.at[pl.ds(NS * SB, REM)])
        pltpu.sync_copy(obuf.at[pl.ds(0, REM)], o_sp.at[pl.ds(NS * SB, REM)])
    plsc.subcore_barrier()

    def _scale(gb, sb_, q):
        for t in range(CH // 16):
            vvec = vals_vm[q, pl.ds(t * 16, 16)]
            base = t * 16
            for lane in range(16):
                v = vvec[lane]
                for j in range(DH // 16):
                    sl = pl.ds(j * 16, 16)
                    sb_[base + lane, sl] = gb[base + lane, sl] * v

    def _hop(gather_issue, gather_wait, scat_ref, scat_vm):
        def blk_body(blk, _):
            pltpu.sync_copy(rows_hbm.at[s, blk], rows_vm)
            pltpu.sync_copy(cols_hbm.at[s, blk], cols_vm)
            pltpu.sync_copy(vals_hbm.at[s, blk], vals_vm)
            for b in (0, 1):
                gather_issue(b, b)
            def body(t, _):
                for b in (0, 1):
                    q = 2 * t + b
                    gather_wait(b)
                    @pl.when(t > 0)
                    def _():
                        pltpu.make_async_copy(
                            sbuf[b], scat_ref.at[scat_vm.at[q]],
                            ssem[b]).wait()
                    _scale(gbuf[b], sbuf[b], q)
                    pltpu.async_copy(
                        sbuf[b], scat_ref.at[scat_vm.at[q]], ssem[b],
                        add=True)
                    @pl.when(t < CPB // 2 - 1)
                    def _():
                        gather_issue(q + 2, b)
                return 0
            lax.fori_loop(0, CPB // 2, body, 0)
            for b in (0, 1):
                q = CPB - 2 + b
                pltpu.make_async_copy(
                    sbuf[b], scat_ref.at[scat_vm.at[q]], ssem[b]).wait()
            return 0
        lax.fori_loop(0, NB, blk_body, 0)

    def h1_issue(q, b):
        for j in range(CH // 16):
            sl = pl.ds(j * 16, 16)
            ibuf[b][sl] = rows_vm[q, sl] + cN
        pltpu.async_copy(embs2_hbm.at[ibuf[b]], gbuf[b], gsem[b])
    def h1_wait(b):
        pltpu.make_async_copy(embs2_hbm.at[ibuf[b]], gbuf[b], gsem[b]).wait()
    _hop(h1_issue, h1_wait, h_sp, cols_vm)
    plsc.subcore_barrier()

    def h2_issue(q, b):
        pltpu.async_copy(h_sp.at[cols_vm.at[q]], gbuf[b], gsem[b])
    def h2_wait(b):
        pltpu.make_async_copy(h_sp.at[cols_vm.at[0]], gbuf[b], gsem[b]).wait()
    _hop(h2_issue, h2_wait, o_sp, rows_vm)
    plsc.subcore_barrier()

    def _leaky(nrows):
        def lbody(i, _):
            for j in range(DH // 16):
                sl = pl.ds(j * 16, 16)
                x = obuf[i, sl]
                obuf[i, sl] = jnp.where(x >= 0, x, x * LEAKY)
            return 0
        lax.fori_loop(0, nrows, lbody, 0)

    for k in range(SB // OB):
        ro = rbase + k * OB
        pltpu.sync_copy(o_sp.at[pl.ds(ro, OB)], obuf)
        _leaky(OB)
        pltpu.sync_copy(obuf, out2_hbm.at[pl.ds(cN + ro, OB)])
    @pl.when(s == NS - 1)
    def _():
        pltpu.sync_copy(o_sp.at[pl.ds(NS * SB, REM)], obuf.at[pl.ds(0, REM)])
        _leaky(REM)
        pltpu.sync_copy(obuf.at[pl.ds(0, REM)],
                        out2_hbm.at[pl.ds(cN + NS * SB, REM)])


@jax.jit
def _hgcn_sc(rows, cols, vals, embs2):
    mesh = plsc.VectorSubcoreMesh(core_axis_name="c", subcore_axis_name="s")
    return pl.kernel(
        _hgcn_body,
        out_type=jax.ShapeDtypeStruct((2 * N, DH), jnp.float32),
        mesh=mesh,
        scratch_types=[
            pltpu.VMEM_SHARED((N, DH), jnp.float32),
            pltpu.VMEM_SHARED((N, DH), jnp.float32),
            pltpu.VMEM((CPB, CH), jnp.int32),
            pltpu.VMEM((CPB, CH), jnp.int32),
            pltpu.VMEM((CPB, CH), jnp.float32),
            pltpu.VMEM((CH,), jnp.int32),
            pltpu.VMEM((CH,), jnp.int32),
            pltpu.VMEM((CH, DH), jnp.float32),
            pltpu.VMEM((CH, DH), jnp.float32),
            pltpu.VMEM((CH, DH), jnp.float32),
            pltpu.VMEM((CH, DH), jnp.float32),
            pltpu.VMEM((OB, DH), jnp.float32),
            pltpu.SemaphoreType.DMA,
            pltpu.SemaphoreType.DMA,
            pltpu.SemaphoreType.DMA,
            pltpu.SemaphoreType.DMA,
        ],
        compiler_params=pltpu.CompilerParams(use_tc_tiling_on_sc=False),
    )(rows, cols, vals, embs2)


def kernel(edge_index, values, embs):
    rows = edge_index[0].astype(jnp.int32).reshape(NS, NB, CPB, CH)
    cols = edge_index[1].astype(jnp.int32).reshape(NS, NB, CPB, CH)
    vals = values.reshape(NS, NB, CPB, CH)
    embs2 = jnp.concatenate([embs[:, :DH], embs[:, DH:]], axis=0)
    out2 = _hgcn_sc(rows, cols, vals, embs2)
    return jnp.concatenate([out2[:N], out2[N:]], axis=1)


# speedup vs baseline: 10.0125x; 1.1086x over previous
import functools

import jax
import jax.numpy as jnp
from jax import lax
from jax.experimental import pallas as pl
from jax.experimental.pallas import tpu as pltpu
from jax.experimental.pallas import tpu_sc as plsc

N = 10000
E = 320000
D = 128
DH = D // 2
LEAKY = 0.2

NS = 16
CH = 80
EPT = E // NS
NCHUNK = EPT // CH
CPB = 50
NB = NCHUNK // CPB
SB = 624
REM = N - NS * SB
OB = 48


def _hgcn_body(rows_hbm, cols_hbm, vals_hbm, embs2_hbm, out2_hbm,
               h_sp, o_sp,
               rows_vm, cols_vm, vals_vm,
               ib0, ib1, gb0, gb1, sb0, sb1, obuf,
               gsem0, gsem1, ssem0, ssem1):
    c = lax.axis_index("c")
    s = lax.axis_index("s")
    cN = c * N
    ibuf = (ib0, ib1)
    gbuf = (gb0, gb1)
    sbuf = (sb0, sb1)
    gsem = (gsem0, gsem1)
    ssem = (ssem0, ssem1)

    def zbody(i, _):
        zero = jnp.zeros((16,), jnp.float32)
        for j in range(DH // 16):
            obuf[i, pl.ds(j * 16, 16)] = zero
        return 0
    lax.fori_loop(0, OB, zbody, 0)
    rbase = s * SB
    for k in range(SB // OB):
        pltpu.sync_copy(obuf, h_sp.at[pl.ds(rbase + k * OB, OB)])
        pltpu.sync_copy(obuf, o_sp.at[pl.ds(rbase + k * OB, OB)])
    @pl.when(s == NS - 1)
    def _():
        pltpu.sync_copy(obuf.at[pl.ds(0, REM)], h_sp.at[pl.ds(NS * SB, REM)])
        pltpu.sync_copy(obuf.at[pl.ds(0, REM)], o_sp.at[pl.ds(NS * SB, REM)])
    plsc.subcore_barrier()

    def _scale(gb, sb_, q):
        for t in range(CH // 16):
            vvec = vals_vm[q, pl.ds(t * 16, 16)]
            base = t * 16
            for lane in range(16):
                v = vvec[lane]
                for j in range(DH // 16):
                    sl = pl.ds(j * 16, 16)
                    sb_[base + lane, sl] = gb[base + lane, sl] * v

    def _hop(gather_issue, gather_wait, scat_ref, scat_vm):
        def blk_body(blk, _):
            pltpu.sync_copy(rows_hbm.at[s, blk], rows_vm)
            pltpu.sync_copy(cols_hbm.at[s, blk], cols_vm)
            pltpu.sync_copy(vals_hbm.at[s, blk], vals_vm)
            for b in (0, 1):
                gather_issue(b, b)
            def body(t, _):
                for b in (0, 1):
                    q = 2 * t + b
                    gather_wait(b)
                    @pl.when(t > 0)
                    def _():
                        pltpu.make_async_copy(
                            sbuf[b], scat_ref.at[scat_vm.at[q]],
                            ssem[b]).wait()
                    _scale(gbuf[b], sbuf[b], q)
                    pltpu.async_copy(
                        sbuf[b], scat_ref.at[scat_vm.at[q]], ssem[b],
                        add=True)
                    @pl.when(t < CPB // 2 - 1)
                    def _():
                        gather_issue(q + 2, b)
                return 0
            lax.fori_loop(0, CPB // 2, body, 0)
            for b in (0, 1):
                q = CPB - 2 + b
                pltpu.make_async_copy(
                    sbuf[b], scat_ref.at[scat_vm.at[q]], ssem[b]).wait()
            return 0
        lax.fori_loop(0, NB, blk_body, 0)

    def h1_issue(q, b):
        for j in range(CH // 16):
            sl = pl.ds(j * 16, 16)
            ibuf[b][sl] = rows_vm[q, sl] * 2 + c
        pltpu.async_copy(embs2_hbm.at[ibuf[b]], gbuf[b], gsem[b])
    def h1_wait(b):
        pltpu.make_async_copy(embs2_hbm.at[ibuf[b]], gbuf[b], gsem[b]).wait()
    _hop(h1_issue, h1_wait, h_sp, cols_vm)
    plsc.subcore_barrier()

    def h2_issue(q, b):
        pltpu.async_copy(h_sp.at[cols_vm.at[q]], gbuf[b], gsem[b])
    def h2_wait(b):
        pltpu.make_async_copy(h_sp.at[cols_vm.at[0]], gbuf[b], gsem[b]).wait()
    _hop(h2_issue, h2_wait, o_sp, rows_vm)
    plsc.subcore_barrier()

    def _leaky(nrows):
        def lbody(i, _):
            for j in range(DH // 16):
                sl = pl.ds(j * 16, 16)
                x = obuf[i, sl]
                obuf[i, sl] = jnp.where(x >= 0, x, x * LEAKY)
            return 0
        lax.fori_loop(0, nrows, lbody, 0)

    csl = pl.ds(c * DH, DH)
    for k in range(SB // OB):
        ro = rbase + k * OB
        pltpu.sync_copy(o_sp.at[pl.ds(ro, OB)], obuf)
        _leaky(OB)
        pltpu.sync_copy(obuf, out2_hbm.at[pl.ds(ro, OB), csl])
    @pl.when(s == NS - 1)
    def _():
        pltpu.sync_copy(o_sp.at[pl.ds(NS * SB, REM)], obuf.at[pl.ds(0, REM)])
        _leaky(REM)
        pltpu.sync_copy(obuf.at[pl.ds(0, REM)],
                        out2_hbm.at[pl.ds(NS * SB, REM), csl])


@jax.jit
def _hgcn_sc(rows, cols, vals, embs2):
    mesh = plsc.VectorSubcoreMesh(core_axis_name="c", subcore_axis_name="s")
    return pl.kernel(
        _hgcn_body,
        out_type=jax.ShapeDtypeStruct((N, D), jnp.float32),
        mesh=mesh,
        scratch_types=[
            pltpu.VMEM_SHARED((N, DH), jnp.float32),
            pltpu.VMEM_SHARED((N, DH), jnp.float32),
            pltpu.VMEM((CPB, CH), jnp.int32),
            pltpu.VMEM((CPB, CH), jnp.int32),
            pltpu.VMEM((CPB, CH), jnp.float32),
            pltpu.VMEM((CH,), jnp.int32),
            pltpu.VMEM((CH,), jnp.int32),
            pltpu.VMEM((CH, DH), jnp.float32),
            pltpu.VMEM((CH, DH), jnp.float32),
            pltpu.VMEM((CH, DH), jnp.float32),
            pltpu.VMEM((CH, DH), jnp.float32),
            pltpu.VMEM((OB, DH), jnp.float32),
            pltpu.SemaphoreType.DMA,
            pltpu.SemaphoreType.DMA,
            pltpu.SemaphoreType.DMA,
            pltpu.SemaphoreType.DMA,
        ],
        compiler_params=pltpu.CompilerParams(use_tc_tiling_on_sc=False),
    )(rows, cols, vals, embs2)


def kernel(edge_index, values, embs):
    rows = edge_index[0].astype(jnp.int32).reshape(NS, NB, CPB, CH)
    cols = edge_index[1].astype(jnp.int32).reshape(NS, NB, CPB, CH)
    vals = values.reshape(NS, NB, CPB, CH)
    embs2 = embs.reshape(2 * N, DH)
    return _hgcn_sc(rows, cols, vals, embs2)
